# Initial kernel scaffold; baseline (speedup 1.0000x reference)
import jax
import jax.numpy as jnp
from jax.experimental import pallas as pl


def kernel(x_user, x_item, params_ui, params_iu, edge_index_ui, edge_index_iu):
    raise NotImplementedError("write your pallas kernel here")


# speedup vs baseline: 3.9594x; 3.9594x over previous
import jax
import jax.numpy as jnp
from jax import lax
from jax.experimental import pallas as pl
from jax.experimental.pallas import tpu as pltpu
from jax.experimental.pallas import tpu_sc as plsc

N = 10000
D = 256
H = 256
E = 160000

QTR = 64
TILES = 16
CH = 128
SCH = 5
SUP = CH * SCH
NSUP = E // SUP
STRIPE = N // TILES
ZR = STRIPE // 5


def _sc_phase(tid, x_ref, es_ref, ed_ref, sums_ref, cnt_ref, do_cnt,
              acc_sh, rows_v, sidx_v, didx_v, cnt_v, zrows_ref, zcnt_ref,
              sem_g, sem_s):
    pltpu.sync_copy(zrows_ref, rows_v.at[pl.ds(0, ZR)])
    for z in range(5):
        pltpu.sync_copy(rows_v.at[pl.ds(0, ZR)],
                        acc_sh.at[pl.ds(tid * STRIPE + z * ZR, ZR)])
    if do_cnt:
        pltpu.sync_copy(zcnt_ref, cnt_v)
    plsc.subcore_barrier()

    nsc = (NSUP - tid + TILES - 1) // TILES
    ones = jnp.ones((16,), jnp.float32)

    def body(i, carry):
        s = tid + i * TILES
        row0 = s * SCH
        pltpu.sync_copy(es_ref.at[pl.ds(row0, SCH)], sidx_v)
        pltpu.sync_copy(ed_ref.at[pl.ds(row0, SCH)], didx_v)
        gathers = [
            pltpu.async_copy(x_ref.at[sidx_v.at[j]],
                             rows_v.at[pl.ds(j * CH, CH)], sem_g)
            for j in range(SCH)
        ]
        for g in gathers:
            g.wait()
        scatters = [
            pltpu.async_copy(rows_v.at[pl.ds(j * CH, CH)],
                             acc_sh.at[didx_v.at[j]], sem_s, add=True)
            for j in range(SCH)
        ]
        if do_cnt:
            for j in range(SCH):
                for k in range(CH // 16):
                    dv = didx_v[j, pl.ds(k * 16, 16)]
                    plsc.addupdate_scatter(cnt_v, [dv], ones)
        for sc in scatters:
            sc.wait()
        return carry

    lax.fori_loop(0, nsc, body, 0)
    plsc.subcore_barrier()

    for z in range(5):
        r0 = tid * STRIPE + z * ZR
        pltpu.sync_copy(acc_sh.at[pl.ds(r0, ZR)], rows_v.at[pl.ds(0, ZR)])
        pltpu.sync_copy(rows_v.at[pl.ds(0, ZR)],
                        sums_ref.at[pl.ds(r0, ZR)])
    if do_cnt:
        pltpu.sync_copy(cnt_v, cnt_ref.at[tid])
    plsc.subcore_barrier()


def _sc_body(xu0, xu1, xu2, xu3, xi0, xi1, xi2, xi3,
             es_ui, ed_ui, es_iu, ed_iu, zrows, zcnt,
             s_ui, s_iu, cnt_ui, cnt_iu,
             acc_sh, rows_v, sidx_v, didx_v, cnt_v, sem_g, sem_s):
    cid = lax.axis_index("c")
    tid = lax.axis_index("s")
    common = dict(acc_sh=acc_sh, rows_v=rows_v, sidx_v=sidx_v,
                  didx_v=didx_v, cnt_v=cnt_v, zrows_ref=zrows,
                  zcnt_ref=zcnt, sem_g=sem_g, sem_s=sem_s)
    xu = (xu0, xu1, xu2, xu3)
    xi = (xi0, xi1, xi2, xi3)

    def run(c, xq, es, ed, s_out, cnt_out, q):
        @pl.when(cid == c)
        def _():
            _sc_phase(tid, xq[2 * q + c], es, ed, s_out.at[2 * q + c],
                      cnt_out, do_cnt=(q == 0 and c == 0), **common)

    for (xq, es, ed, s_out, cnt_out) in (
            (xu, es_ui, ed_ui, s_ui, cnt_ui),
            (xi, es_iu, ed_iu, s_iu, cnt_iu)):
        for q in range(2):
            run(0, xq, es, ed, s_out, cnt_out, q)
            run(1, xq, es, ed, s_out, cnt_out, q)


_sc_aggregate = pl.kernel(
    _sc_body,
    out_type=[
        jax.ShapeDtypeStruct((4, N, QTR), jnp.float32),
        jax.ShapeDtypeStruct((4, N, QTR), jnp.float32),
        jax.ShapeDtypeStruct((TILES, N), jnp.float32),
        jax.ShapeDtypeStruct((TILES, N), jnp.float32),
    ],
    mesh=plsc.VectorSubcoreMesh(core_axis_name="c", subcore_axis_name="s"),
    compiler_params=pltpu.CompilerParams(use_tc_tiling_on_sc=False,
                                         needs_layout_passes=False),
    scratch_types=[
        pltpu.VMEM_SHARED((N, QTR), jnp.float32),
        pltpu.VMEM((SUP, QTR), jnp.float32),
        pltpu.VMEM((SCH, CH), jnp.int32),
        pltpu.VMEM((SCH, CH), jnp.int32),
        pltpu.VMEM((N,), jnp.float32),
        pltpu.SemaphoreType.DMA,
        pltpu.SemaphoreType.DMA,
    ],
)


BLK = 1000


def _dense_body(xu_ref, xi_ref, su_ref, si_ref,
                cu_ref, ci_ref, Wu_ref, bu_ref, Wi_ref, bi_ref,
                ou_ref, oi_ref):
    def one_type(x_tgt, s4, cnt16, W, b):
        cnt = jnp.maximum(jnp.sum(cnt16, axis=1), 1.0)
        sums = jnp.concatenate([s4[0], s4[1], s4[2], s4[3]], axis=1)
        mean = sums / cnt[:, None]
        dot = lambda a, w: jnp.dot(a, w, preferred_element_type=jnp.float32)
        agg = dot(mean, W[0])
        tgt = dot(x_tgt, W[1])
        hr = dot(agg, W[5]) + b[0]
        hn = dot(agg, W[6]) + b[3]

        def cell(inp):
            r = jax.nn.sigmoid(dot(inp, W[2]) + hr)
            z = jax.nn.sigmoid(dot(inp + agg, W[3]) + b[1])
            n = jnp.tanh(dot(inp, W[4]) + b[2] + r * hn)
            return z * n + (1.0 - z) * agg

        return jnp.maximum(cell(cell(tgt)), 0.0)

    oi_ref[...] = one_type(xi_ref[...], su_ref[...], cu_ref[...],
                           Wu_ref[...], bu_ref[...])
    ou_ref[...] = one_type(xu_ref[...], si_ref[...], ci_ref[...],
                           Wi_ref[...], bi_ref[...])


_dense = pl.pallas_call(
    _dense_body,
    grid=(N // BLK,),
    in_specs=[
        pl.BlockSpec((BLK, D), lambda i: (i, 0)),
        pl.BlockSpec((BLK, D), lambda i: (i, 0)),
        pl.BlockSpec((4, BLK, QTR), lambda i: (0, i, 0)),
        pl.BlockSpec((4, BLK, QTR), lambda i: (0, i, 0)),
        pl.BlockSpec((BLK, TILES), lambda i: (i, 0)),
        pl.BlockSpec((BLK, TILES), lambda i: (i, 0)),
        pl.BlockSpec((7, H, H), lambda i: (0, 0, 0)),
        pl.BlockSpec((4, H), lambda i: (0, 0)),
        pl.BlockSpec((7, H, H), lambda i: (0, 0, 0)),
        pl.BlockSpec((4, H), lambda i: (0, 0)),
    ],
    out_specs=[
        pl.BlockSpec((BLK, H), lambda i: (i, 0)),
        pl.BlockSpec((BLK, H), lambda i: (i, 0)),
    ],
    out_shape=[
        jax.ShapeDtypeStruct((N, H), jnp.float32),
        jax.ShapeDtypeStruct((N, H), jnp.float32),
    ],
)


def _wpack(p):
    g = p['gru']
    W = jnp.stack([p['W_src'], p['W_tgt'], g['Wir'], g['Whz'], g['Win'],
                   g['Whr'], g['Whn']])
    b = jnp.stack([g['bir'] + g['bhr'], g['biz'] + g['bhz'], g['bin'],
                   g['bhn']])
    return W, b


def kernel(x_user, x_item, params_ui, params_iu, edge_index_ui,
           edge_index_iu):
    xu = [x_user[:, QTR * k:QTR * (k + 1)] for k in range(4)]
    xi = [x_item[:, QTR * k:QTR * (k + 1)] for k in range(4)]
    es_ui = edge_index_ui[0].reshape(E // CH, CH)
    ed_ui = edge_index_ui[1].reshape(E // CH, CH)
    es_iu = edge_index_iu[0].reshape(E // CH, CH)
    ed_iu = edge_index_iu[1].reshape(E // CH, CH)
    zrows = jnp.zeros((ZR, QTR), jnp.float32)
    zcnt = jnp.zeros((N,), jnp.float32)

    s_ui, s_iu, cnt_ui, cnt_iu = _sc_aggregate(
        *xu, *xi, es_ui, ed_ui, es_iu, ed_iu, zrows, zcnt)

    W_ui, b_ui = _wpack(params_ui)
    W_iu, b_iu = _wpack(params_iu)
    out_user, out_item = _dense(x_user, x_item, s_ui, s_iu,
                                cnt_ui.T, cnt_iu.T, W_ui, b_ui, W_iu, b_iu)
    return out_user, out_item
